# Initial kernel scaffold; baseline (speedup 1.0000x reference)
import jax
import jax.numpy as jnp
from jax.experimental import pallas as pl


def kernel(xyz_target, xyz_src, feat_target, feat_src, W1, b1, W2, b2):
    raise NotImplementedError("write your pallas kernel here")


# speedup vs baseline: 26.0300x; 26.0300x over previous
import functools

import jax
import jax.numpy as jnp
from jax.experimental import pallas as pl


def _fused_body(xyz_t_ref, xyz_s_ref, feat_t_ref, feat_s_ref,
                w1a_ref, w1b_ref, b1_ref, w2_ref, b2_ref, out_ref):
    xyz_t = xyz_t_ref[0]
    xyz_s = xyz_s_ref[0]
    bn = xyz_t.shape[0]
    m = xyz_s.shape[0]

    sq_t = jnp.sum(xyz_t * xyz_t, axis=1, keepdims=True)
    sq_s = jnp.sum(xyz_s * xyz_s, axis=1, keepdims=True).T
    cross = jax.lax.dot_general(
        xyz_t, xyz_s, (((1,), (1,)), ((), ())),
        preferred_element_type=jnp.float32)
    d2 = jnp.maximum(sq_t + sq_s - 2.0 * cross, 0.0)

    iota_m = jax.lax.broadcasted_iota(jnp.int32, (bn, m), 1)
    inf = jnp.float32(jnp.inf)
    sparse_w = jnp.zeros((bn, m), dtype=jnp.float32)
    total = jnp.zeros((bn, 1), dtype=jnp.float32)
    for _ in range(3):
        mk = jnp.min(d2, axis=1, keepdims=True)
        ik = jnp.min(jnp.where(d2 == mk, iota_m, m),
                     axis=1, keepdims=True)
        hit = iota_m == ik
        rk = 1.0 / (mk + 1e-8)
        sparse_w = sparse_w + jnp.where(hit, rk, 0.0)
        total = total + rk
        d2 = jnp.where(hit, inf, d2)
    sparse_w = sparse_w / total

    interp = jax.lax.dot_general(
        sparse_w, feat_s_ref[0], (((1,), (0,)), ((), ())),
        preferred_element_type=jnp.float32)

    h = jax.nn.relu(
        jax.lax.dot_general(interp, w1a_ref[...], (((1,), (0,)), ((), ())),
                            preferred_element_type=jnp.float32)
        + jax.lax.dot_general(feat_t_ref[0], w1b_ref[...], (((1,), (0,)), ((), ())),
                              preferred_element_type=jnp.float32)
        + b1_ref[...])
    out_ref[0] = (
        jax.lax.dot_general(h, w2_ref[...], (((1,), (0,)), ((), ())),
                            preferred_element_type=jnp.float32)
        + b2_ref[...])


@functools.partial(jax.jit, static_argnames=("bn",))
def _fused(xyz_target, xyz_src, feat_target, feat_src, W1a, W1b, b1, W2, b2,
           bn=512):
    B, N, _ = xyz_target.shape
    M = xyz_src.shape[1]
    C1 = feat_target.shape[2]
    C2 = feat_src.shape[2]
    Cout = W2.shape[1]
    grid = (B, N // bn)
    return pl.pallas_call(
        _fused_body,
        grid=grid,
        in_specs=[
            pl.BlockSpec((1, bn, 3), lambda b, i: (b, i, 0)),
            pl.BlockSpec((1, M, 3), lambda b, i: (b, 0, 0)),
            pl.BlockSpec((1, bn, C1), lambda b, i: (b, i, 0)),
            pl.BlockSpec((1, M, C2), lambda b, i: (b, 0, 0)),
            pl.BlockSpec((C2, Cout), lambda b, i: (0, 0)),
            pl.BlockSpec((C1, Cout), lambda b, i: (0, 0)),
            pl.BlockSpec((1, Cout), lambda b, i: (0, 0)),
            pl.BlockSpec((Cout, Cout), lambda b, i: (0, 0)),
            pl.BlockSpec((1, Cout), lambda b, i: (0, 0)),
        ],
        out_specs=pl.BlockSpec((1, bn, Cout), lambda b, i: (b, i, 0)),
        out_shape=jax.ShapeDtypeStruct((B, N, Cout), jnp.float32),
    )(xyz_target, xyz_src, feat_target, feat_src, W1a, W1b, b1, W2, b2)


def kernel(xyz_target, xyz_src, feat_target, feat_src, W1, b1, W2, b2):
    C2 = feat_src.shape[2]
    W1a = W1[:C2]
    W1b = W1[C2:]
    return _fused(xyz_target, xyz_src, feat_target, feat_src,
                  W1a, W1b, b1.reshape(1, -1), W2, b2.reshape(1, -1))


# speedup vs baseline: 28.9300x; 1.1114x over previous
import functools

import jax
import jax.numpy as jnp
from jax.experimental import pallas as pl


def _fused_body(xyz_t_ref, xyz_s_ref, feat_t_ref, feat_s_ref,
                w1a_ref, w1b_ref, b1_ref, w2_ref, b2_ref, out_ref):
    xyz_t = xyz_t_ref[0]
    xyz_s = xyz_s_ref[0]
    bn = xyz_t.shape[0]
    m = xyz_s.shape[0]

    sq_t = jnp.sum(xyz_t * xyz_t, axis=1, keepdims=True)
    sq_s = jnp.sum(xyz_s * xyz_s, axis=1, keepdims=True).T
    cross = jax.lax.dot_general(
        xyz_t, xyz_s, (((1,), (1,)), ((), ())),
        preferred_element_type=jnp.float32)
    d2 = jnp.maximum(sq_t + sq_s - 2.0 * cross, 0.0)

    iota_m = jax.lax.broadcasted_iota(jnp.int32, (bn, m), 1)
    key = (jax.lax.bitcast_convert_type(d2, jnp.int32) & ~(m - 1)) | iota_m
    big = jnp.int32(0x7FFFFFFF)
    sparse_w = jnp.zeros((bn, m), dtype=jnp.float32)
    total = jnp.zeros((bn, 1), dtype=jnp.float32)
    for _ in range(3):
        mk = jnp.min(key, axis=1, keepdims=True)
        dk = jax.lax.bitcast_convert_type(mk & ~(m - 1), jnp.float32)
        rk = 1.0 / (dk + 1e-8)
        hit = key == mk
        sparse_w = sparse_w + jnp.where(hit, rk, 0.0)
        total = total + rk
        key = jnp.where(hit, big, key)
    sparse_w = sparse_w / total

    interp = jax.lax.dot_general(
        sparse_w, feat_s_ref[0], (((1,), (0,)), ((), ())),
        preferred_element_type=jnp.float32)

    h = jax.nn.relu(
        jax.lax.dot_general(interp, w1a_ref[...], (((1,), (0,)), ((), ())),
                            preferred_element_type=jnp.float32)
        + jax.lax.dot_general(feat_t_ref[0], w1b_ref[...], (((1,), (0,)), ((), ())),
                              preferred_element_type=jnp.float32)
        + b1_ref[...])
    out_ref[0] = (
        jax.lax.dot_general(h, w2_ref[...], (((1,), (0,)), ((), ())),
                            preferred_element_type=jnp.float32)
        + b2_ref[...])


@functools.partial(jax.jit, static_argnames=("bn",))
def _fused(xyz_target, xyz_src, feat_target, feat_src, W1a, W1b, b1, W2, b2,
           bn=512):
    B, N, _ = xyz_target.shape
    M = xyz_src.shape[1]
    C1 = feat_target.shape[2]
    C2 = feat_src.shape[2]
    Cout = W2.shape[1]
    grid = (B, N // bn)
    return pl.pallas_call(
        _fused_body,
        grid=grid,
        in_specs=[
            pl.BlockSpec((1, bn, 3), lambda b, i: (b, i, 0)),
            pl.BlockSpec((1, M, 3), lambda b, i: (b, 0, 0)),
            pl.BlockSpec((1, bn, C1), lambda b, i: (b, i, 0)),
            pl.BlockSpec((1, M, C2), lambda b, i: (b, 0, 0)),
            pl.BlockSpec((C2, Cout), lambda b, i: (0, 0)),
            pl.BlockSpec((C1, Cout), lambda b, i: (0, 0)),
            pl.BlockSpec((1, Cout), lambda b, i: (0, 0)),
            pl.BlockSpec((Cout, Cout), lambda b, i: (0, 0)),
            pl.BlockSpec((1, Cout), lambda b, i: (0, 0)),
        ],
        out_specs=pl.BlockSpec((1, bn, Cout), lambda b, i: (b, i, 0)),
        out_shape=jax.ShapeDtypeStruct((B, N, Cout), jnp.float32),
    )(xyz_target, xyz_src, feat_target, feat_src, W1a, W1b, b1, W2, b2)


def kernel(xyz_target, xyz_src, feat_target, feat_src, W1, b1, W2, b2):
    C2 = feat_src.shape[2]
    W1a = W1[:C2]
    W1b = W1[C2:]
    return _fused(xyz_target, xyz_src, feat_target, feat_src,
                  W1a, W1b, b1.reshape(1, -1), W2, b2.reshape(1, -1))


# speedup vs baseline: 33.4290x; 1.1555x over previous
import functools

import jax
import jax.numpy as jnp
from jax.experimental import pallas as pl


def _fused_body(xyz_t_ref, xyz_s_ref, feat_t_ref, feat_s_ref,
                w1a_ref, w1b_ref, b1_ref, w2_ref, b2_ref, out_ref):
    xyz_t = xyz_t_ref[0]
    xyz_s = xyz_s_ref[0]
    bn = xyz_t.shape[0]
    m = xyz_s.shape[0]

    sq_t = jnp.sum(xyz_t * xyz_t, axis=1, keepdims=True)
    sq_s = jnp.sum(xyz_s * xyz_s, axis=1, keepdims=True).T
    cross = jax.lax.dot_general(
        xyz_t, xyz_s, (((1,), (1,)), ((), ())),
        preferred_element_type=jnp.float32)
    d2 = jnp.maximum(sq_t + sq_s - 2.0 * cross, 0.0)

    iota_m = jax.lax.broadcasted_iota(jnp.int32, (bn, m), 1)
    key = jax.lax.bitcast_convert_type(
        (jax.lax.bitcast_convert_type(d2, jnp.int32) & ~(m - 1)) | iota_m,
        jnp.float32)
    big = jnp.float32(3.4e38)
    m1 = jnp.min(key, axis=1, keepdims=True)
    h1 = key == m1
    m2 = jnp.min(jnp.where(h1, big, key), axis=1, keepdims=True)
    h2 = key == m2
    m3 = jnp.min(jnp.where(h1 | h2, big, key), axis=1, keepdims=True)

    def unpack_rk(mk):
        dk = jax.lax.bitcast_convert_type(
            jax.lax.bitcast_convert_type(mk, jnp.int32) & ~(m - 1), jnp.float32)
        return 1.0 / (dk + 1e-8)

    r1, r2, r3 = unpack_rk(m1), unpack_rk(m2), unpack_rk(m3)
    total = r1 + r2 + r3
    w1n, w2n, w3n = r1 / total, r2 / total, r3 / total
    zero = jnp.zeros((bn, m), dtype=jnp.float32)
    sparse_w = (jnp.where(h1, w1n, zero)
                + jnp.where(h2, w2n, zero)
                + jnp.where(key == m3, w3n, zero))

    interp = jax.lax.dot_general(
        sparse_w, feat_s_ref[0], (((1,), (0,)), ((), ())),
        preferred_element_type=jnp.float32)

    h = jax.nn.relu(
        jax.lax.dot_general(interp, w1a_ref[...], (((1,), (0,)), ((), ())),
                            preferred_element_type=jnp.float32)
        + jax.lax.dot_general(feat_t_ref[0], w1b_ref[...], (((1,), (0,)), ((), ())),
                              preferred_element_type=jnp.float32)
        + b1_ref[...])
    out_ref[0] = (
        jax.lax.dot_general(h, w2_ref[...], (((1,), (0,)), ((), ())),
                            preferred_element_type=jnp.float32)
        + b2_ref[...])


@functools.partial(jax.jit, static_argnames=("bn",))
def _fused(xyz_target, xyz_src, feat_target, feat_src, W1a, W1b, b1, W2, b2,
           bn=512):
    B, N, _ = xyz_target.shape
    M = xyz_src.shape[1]
    C1 = feat_target.shape[2]
    C2 = feat_src.shape[2]
    Cout = W2.shape[1]
    grid = (B, N // bn)
    return pl.pallas_call(
        _fused_body,
        grid=grid,
        in_specs=[
            pl.BlockSpec((1, bn, 3), lambda b, i: (b, i, 0)),
            pl.BlockSpec((1, M, 3), lambda b, i: (b, 0, 0)),
            pl.BlockSpec((1, bn, C1), lambda b, i: (b, i, 0)),
            pl.BlockSpec((1, M, C2), lambda b, i: (b, 0, 0)),
            pl.BlockSpec((C2, Cout), lambda b, i: (0, 0)),
            pl.BlockSpec((C1, Cout), lambda b, i: (0, 0)),
            pl.BlockSpec((1, Cout), lambda b, i: (0, 0)),
            pl.BlockSpec((Cout, Cout), lambda b, i: (0, 0)),
            pl.BlockSpec((1, Cout), lambda b, i: (0, 0)),
        ],
        out_specs=pl.BlockSpec((1, bn, Cout), lambda b, i: (b, i, 0)),
        out_shape=jax.ShapeDtypeStruct((B, N, Cout), jnp.float32),
    )(xyz_target, xyz_src, feat_target, feat_src, W1a, W1b, b1, W2, b2)


def kernel(xyz_target, xyz_src, feat_target, feat_src, W1, b1, W2, b2):
    C2 = feat_src.shape[2]
    W1a = W1[:C2]
    W1b = W1[C2:]
    return _fused(xyz_target, xyz_src, feat_target, feat_src,
                  W1a, W1b, b1.reshape(1, -1), W2, b2.reshape(1, -1))
